# Initial kernel scaffold; baseline (speedup 1.0000x reference)
import jax
import jax.numpy as jnp
from jax.experimental import pallas as pl


def kernel(pred, target):
    raise NotImplementedError("write your pallas kernel here")


# speedup vs baseline: 10.3351x; 10.3351x over previous
import functools

import jax
import jax.numpy as jnp
from jax import lax
from jax.experimental import pallas as pl
from jax.experimental.pallas import tpu as pltpu
from jax.experimental.pallas import tpu_sc as plsc

N = 1_000_000
K = 950_000
NW = 32
C = 31_264
NPAD = NW * C
NB1 = 2048
NB2 = 1024
NB3 = 1024

_mesh = plsc.VectorSubcoreMesh(core_axis_name="c", subcore_axis_name="s")


def _wid():
    return lax.axis_index("s") * 2 + lax.axis_index("c")


def _lane():
    return lax.broadcasted_iota(jnp.int32, (16,), 0)


def _zero_ref(ref, nwords):
    z = jnp.zeros((16,), jnp.int32)

    def body(i, _):
        ref[pl.ds(i * 16, 16)] = z
        return 0

    lax.fori_loop(0, nwords // 16, body, 0)


def _reduce_rows_and_scan(hin, nrows, nbins, kth):
    zeros = jnp.zeros((16,), jnp.int32)
    ones = jnp.ones((16,), jnp.int32)

    def body(g, carry):
        total, b_acc, pre_acc = carry
        acc = zeros
        for r in range(nrows):
            acc = acc + hin[pl.ds(r * nbins + g * 16, 16)]
        cs = plsc.cumsum(acc)
        lt = (total + cs) < kth
        b_acc = b_acc + jnp.sum(jnp.where(lt, ones, zeros))
        pre_acc = pre_acc + jnp.sum(jnp.where(lt, acc, zeros))
        total = total + jnp.sum(acc)
        return total, b_acc, pre_acc

    _, b, pre = lax.fori_loop(
        0, nbins // 16, body,
        (jnp.int32(0), jnp.int32(0), jnp.int32(0)))
    return b, pre


def _reduce_banked(hb, hrow, nbins):
    zeros = jnp.zeros((16,), jnp.int32)

    def body(g, _):
        acc = zeros
        for l in range(16):
            acc = acc + hb[pl.ds(l * nbins + g * 16, 16)]
        hrow[pl.ds(g * 16, 16)] = acc
        return 0

    lax.fori_loop(0, nbins // 16, body, 0)


@functools.partial(
    pl.kernel,
    mesh=_mesh,
    compiler_params=pltpu.CompilerParams(needs_layout_passes=False),
    out_type=(
        jax.ShapeDtypeStruct((NPAD,), jnp.float32),
        jax.ShapeDtypeStruct((NW * NB1,), jnp.int32),
    ),
    scratch_types=[
        pltpu.VMEM((C,), jnp.float32),
        pltpu.VMEM((C,), jnp.float32),
        pltpu.VMEM((16 * NB1,), jnp.int32),
        pltpu.VMEM((NB1,), jnp.int32),
    ],
)
def _k1(p_hbm, t_hbm, err_hbm, h1_hbm, pv, tv, hb, hrow):
    wid = _wid()
    base = wid * C
    pltpu.sync_copy(p_hbm.at[pl.ds(base, C)], pv)
    pltpu.sync_copy(t_hbm.at[pl.ds(base, C)], tv)
    _zero_ref(hb, 16 * NB1)
    lane_off = _lane() * NB1
    ones = jnp.ones((16,), jnp.int32)

    def body(i, _):
        p = pv[pl.ds(i * 16, 16)]
        t = tv[pl.ds(i * 16, 16)]
        e = jnp.abs(p - t)
        pv[pl.ds(i * 16, 16)] = e
        u = lax.bitcast_convert_type(e, jnp.int32)
        off = lane_off + (u >> 20)
        plsc.addupdate_scatter(hb, [off], ones)
        return 0

    lax.fori_loop(0, C // 16, body, 0)
    _reduce_banked(hb, hrow, NB1)
    pltpu.sync_copy(pv, err_hbm.at[pl.ds(base, C)])
    pltpu.sync_copy(hrow, h1_hbm.at[pl.ds(wid * NB1, NB1)])


@functools.partial(
    pl.kernel,
    mesh=_mesh,
    compiler_params=pltpu.CompilerParams(needs_layout_passes=False),
    out_type=(
        jax.ShapeDtypeStruct((NW * NB2,), jnp.int32),
        jax.ShapeDtypeStruct((16,), jnp.int32),
    ),
    scratch_types=[
        pltpu.VMEM((C,), jnp.float32),
        pltpu.VMEM((NW * NB1,), jnp.int32),
        pltpu.VMEM((16 * NB2,), jnp.int32),
        pltpu.VMEM((NB2,), jnp.int32),
        pltpu.VMEM((16,), jnp.int32),
    ],
)
def _k2(err_hbm, h1_hbm, h2_hbm, st1_hbm, ev, hin, hb, hrow, stv):
    wid = _wid()
    base = wid * C
    pltpu.sync_copy(err_hbm.at[pl.ds(base, C)], ev)
    pltpu.sync_copy(h1_hbm, hin)
    b0, pre0 = _reduce_rows_and_scan(hin, NW, NB1, K)
    kp = K - pre0
    _zero_ref(hb, 16 * NB2)
    lane = _lane()
    lane_off = lane * NB2
    ones = jnp.ones((16,), jnp.int32)

    def body(i, _):
        e = ev[pl.ds(i * 16, 16)]
        u = lax.bitcast_convert_type(e, jnp.int32)
        m = (u >> 20) == b0
        off = lane_off + ((u >> 10) & (NB2 - 1))
        plsc.addupdate_scatter(hb, [off], ones, mask=m)
        return 0

    lax.fori_loop(0, C // 16, body, 0)
    _reduce_banked(hb, hrow, NB2)
    pltpu.sync_copy(hrow, h2_hbm.at[pl.ds(wid * NB2, NB2)])

    @pl.when(wid == 0)
    def _():
        zeros = jnp.zeros((16,), jnp.int32)
        stv[...] = jnp.where(lane == 0, b0, jnp.where(lane == 1, kp, zeros))
        pltpu.sync_copy(stv, st1_hbm)


@functools.partial(
    pl.kernel,
    mesh=_mesh,
    compiler_params=pltpu.CompilerParams(needs_layout_passes=False),
    out_type=(
        jax.ShapeDtypeStruct((NW * NB3,), jnp.int32),
        jax.ShapeDtypeStruct((16,), jnp.int32),
    ),
    scratch_types=[
        pltpu.VMEM((C,), jnp.float32),
        pltpu.VMEM((NW * NB2,), jnp.int32),
        pltpu.VMEM((16 * NB3,), jnp.int32),
        pltpu.VMEM((NB3,), jnp.int32),
        pltpu.VMEM((16,), jnp.int32),
    ],
)
def _k3(err_hbm, h2_hbm, st1_hbm, h3_hbm, st2_hbm, ev, hin, hb, hrow, stv):
    wid = _wid()
    base = wid * C
    pltpu.sync_copy(err_hbm.at[pl.ds(base, C)], ev)
    pltpu.sync_copy(h2_hbm, hin)
    pltpu.sync_copy(st1_hbm, stv)
    lane = _lane()
    zeros = jnp.zeros((16,), jnp.int32)
    sv = stv[...]
    b0 = jnp.sum(jnp.where(lane == 0, sv, zeros))
    kp = jnp.sum(jnp.where(lane == 1, sv, zeros))
    b1, pre1 = _reduce_rows_and_scan(hin, NW, NB2, kp)
    p01 = b0 * NB2 + b1
    kpp = kp - pre1
    _zero_ref(hb, 16 * NB3)
    lane_off = lane * NB3
    ones = jnp.ones((16,), jnp.int32)

    def body(i, _):
        e = ev[pl.ds(i * 16, 16)]
        u = lax.bitcast_convert_type(e, jnp.int32)
        m = (u >> 10) == p01
        off = lane_off + (u & (NB3 - 1))
        plsc.addupdate_scatter(hb, [off], ones, mask=m)
        return 0

    lax.fori_loop(0, C // 16, body, 0)
    _reduce_banked(hb, hrow, NB3)
    pltpu.sync_copy(hrow, h3_hbm.at[pl.ds(wid * NB3, NB3)])

    @pl.when(wid == 0)
    def _():
        stv[...] = jnp.where(lane == 0, p01, jnp.where(lane == 1, kpp, zeros))
        pltpu.sync_copy(stv, st2_hbm)


@functools.partial(
    pl.kernel,
    mesh=_mesh,
    compiler_params=pltpu.CompilerParams(needs_layout_passes=False),
    out_type=jax.ShapeDtypeStruct((16,), jnp.float32),
    scratch_types=[
        pltpu.VMEM((NW * NB3,), jnp.int32),
        pltpu.VMEM((16,), jnp.int32),
        pltpu.VMEM((16,), jnp.float32),
    ],
)
def _k4(h3_hbm, st2_hbm, res_hbm, hin, stv, outv):
    wid = _wid()

    @pl.when(wid == 0)
    def _():
        pltpu.sync_copy(h3_hbm, hin)
        pltpu.sync_copy(st2_hbm, stv)
        lane = _lane()
        zeros = jnp.zeros((16,), jnp.int32)
        sv = stv[...]
        p01 = jnp.sum(jnp.where(lane == 0, sv, zeros))
        kpp = jnp.sum(jnp.where(lane == 1, sv, zeros))
        b2, _unused = _reduce_rows_and_scan(hin, NW, NB3, kpp)
        bits = p01 * NB3 + b2
        vec = jnp.zeros((16,), jnp.int32) + bits
        outv[...] = lax.bitcast_convert_type(vec, jnp.float32)
        pltpu.sync_copy(outv, res_hbm)


def kernel(pred, target):
    padp = jnp.concatenate(
        [pred, jnp.full((NPAD - N,), jnp.inf, jnp.float32)])
    padt = jnp.concatenate([target, jnp.zeros((NPAD - N,), jnp.float32)])
    err, h1 = _k1(padp, padt)
    h2, st1 = _k2(err, h1)
    h3, st2 = _k3(err, h2, st1)
    res = _k4(h3, st2)
    return res[0]


# speedup vs baseline: 18.5215x; 1.7921x over previous
import functools

import jax
import jax.numpy as jnp
from jax import lax
from jax.experimental import pallas as pl
from jax.experimental.pallas import tpu as pltpu
from jax.experimental.pallas import tpu_sc as plsc

N = 1_000_000
K = 950_000
NW = 32
C = 31_264
NPAD = NW * C
NB1 = 2048
NB2 = 1024
NB3 = 1024

_mesh = plsc.VectorSubcoreMesh(core_axis_name="c", subcore_axis_name="s")
_params = pltpu.CompilerParams(needs_layout_passes=False)


def _wid():
    return lax.axis_index("s") * 2 + lax.axis_index("c")


def _lane():
    return lax.broadcasted_iota(jnp.int32, (16,), 0)


def _zero_ref(ref, nwords):
    z = jnp.zeros((16,), jnp.int32)

    @plsc.parallel_loop(0, nwords // 16, unroll=8)
    def _(i):
        ref[pl.ds(i * 16, 16)] = z


def _reduce_rows_and_scan(hin, nrows, nbins, kth):
    zeros = jnp.zeros((16,), jnp.int32)
    ones = jnp.ones((16,), jnp.int32)

    @plsc.parallel_loop(
        0, nbins // 16, carry=(jnp.int32(0), jnp.int32(0), jnp.int32(0)))
    def body(g, carry):
        total, b_acc, pre_acc = carry
        acc = zeros
        for r in range(nrows):
            acc = acc + hin[pl.ds(r * nbins + g * 16, 16)]
        cs = plsc.cumsum(acc)
        lt = (total + cs) < kth
        b_acc = b_acc + jnp.sum(jnp.where(lt, ones, zeros))
        pre_acc = pre_acc + jnp.sum(jnp.where(lt, acc, zeros))
        total = total + jnp.sum(acc)
        return total, b_acc, pre_acc

    _, b, pre = body
    return b, pre


def _reduce_banked(hb, hrow, nbins):
    zeros = jnp.zeros((16,), jnp.int32)

    @plsc.parallel_loop(0, nbins // 16, unroll=2)
    def _(g):
        acc = zeros
        for l in range(16):
            acc = acc + hb[pl.ds(l * nbins + g * 16, 16)]
        hrow[pl.ds(g * 16, 16)] = acc


@functools.partial(
    pl.kernel,
    mesh=_mesh,
    compiler_params=_params,
    out_type=(
        jax.ShapeDtypeStruct((NPAD,), jnp.float32),
        jax.ShapeDtypeStruct((NW * NB1,), jnp.int32),
    ),
    scratch_types=[
        pltpu.VMEM((C,), jnp.float32),
        pltpu.VMEM((C,), jnp.float32),
        pltpu.VMEM((16 * NB1,), jnp.int32),
        pltpu.VMEM((NB1,), jnp.int32),
    ],
)
def _k1(p_hbm, t_hbm, err_hbm, h1_hbm, pv, tv, hb, hrow):
    wid = _wid()
    base = wid * C
    pltpu.sync_copy(p_hbm.at[pl.ds(base, C)], pv)
    pltpu.sync_copy(t_hbm.at[pl.ds(base, C)], tv)
    _zero_ref(hb, 16 * NB1)
    lane_off = _lane() * NB1
    ones = jnp.ones((16,), jnp.int32)

    @plsc.parallel_loop(0, C // 16, unroll=8)
    def _(i):
        p = pv[pl.ds(i * 16, 16)]
        t = tv[pl.ds(i * 16, 16)]
        e = jnp.abs(p - t)
        pv[pl.ds(i * 16, 16)] = e
        u = lax.bitcast_convert_type(e, jnp.int32)
        off = lane_off + (u >> 20)
        plsc.addupdate_scatter(hb, [off], ones)

    _reduce_banked(hb, hrow, NB1)
    pltpu.sync_copy(pv, err_hbm.at[pl.ds(base, C)])
    pltpu.sync_copy(hrow, h1_hbm.at[pl.ds(wid * NB1, NB1)])


@functools.partial(
    pl.kernel,
    mesh=_mesh,
    compiler_params=_params,
    out_type=(
        jax.ShapeDtypeStruct((NW * NB2,), jnp.int32),
        jax.ShapeDtypeStruct((16,), jnp.int32),
    ),
    scratch_types=[
        pltpu.VMEM((C,), jnp.float32),
        pltpu.VMEM((NW * NB1,), jnp.int32),
        pltpu.VMEM((16 * NB2,), jnp.int32),
        pltpu.VMEM((NB2,), jnp.int32),
        pltpu.VMEM((16,), jnp.int32),
    ],
)
def _k2(err_hbm, h1_hbm, h2_hbm, st1_hbm, ev, hin, hb, hrow, stv):
    wid = _wid()
    base = wid * C
    pltpu.sync_copy(err_hbm.at[pl.ds(base, C)], ev)
    pltpu.sync_copy(h1_hbm, hin)
    b0, pre0 = _reduce_rows_and_scan(hin, NW, NB1, K)
    kp = K - pre0
    _zero_ref(hb, 16 * NB2)
    lane = _lane()
    lane_off = lane * NB2
    ones = jnp.ones((16,), jnp.int32)

    @plsc.parallel_loop(0, C // 16, unroll=8)
    def _(i):
        e = ev[pl.ds(i * 16, 16)]
        u = lax.bitcast_convert_type(e, jnp.int32)
        m = (u >> 20) == b0
        off = lane_off + ((u >> 10) & (NB2 - 1))
        plsc.addupdate_scatter(hb, [off], ones, mask=m)

    _reduce_banked(hb, hrow, NB2)
    pltpu.sync_copy(hrow, h2_hbm.at[pl.ds(wid * NB2, NB2)])

    @pl.when(wid == 0)
    def _():
        zeros = jnp.zeros((16,), jnp.int32)
        stv[...] = jnp.where(lane == 0, b0, jnp.where(lane == 1, kp, zeros))
        pltpu.sync_copy(stv, st1_hbm)


@functools.partial(
    pl.kernel,
    mesh=_mesh,
    compiler_params=_params,
    out_type=(
        jax.ShapeDtypeStruct((NW * NB3,), jnp.int32),
        jax.ShapeDtypeStruct((16,), jnp.int32),
    ),
    scratch_types=[
        pltpu.VMEM((C,), jnp.float32),
        pltpu.VMEM((NW * NB2,), jnp.int32),
        pltpu.VMEM((16 * NB3,), jnp.int32),
        pltpu.VMEM((NB3,), jnp.int32),
        pltpu.VMEM((16,), jnp.int32),
    ],
)
def _k3(err_hbm, h2_hbm, st1_hbm, h3_hbm, st2_hbm, ev, hin, hb, hrow, stv):
    wid = _wid()
    base = wid * C
    pltpu.sync_copy(err_hbm.at[pl.ds(base, C)], ev)
    pltpu.sync_copy(h2_hbm, hin)
    pltpu.sync_copy(st1_hbm, stv)
    lane = _lane()
    zeros = jnp.zeros((16,), jnp.int32)
    sv = stv[...]
    b0 = jnp.sum(jnp.where(lane == 0, sv, zeros))
    kp = jnp.sum(jnp.where(lane == 1, sv, zeros))
    b1, pre1 = _reduce_rows_and_scan(hin, NW, NB2, kp)
    p01 = b0 * NB2 + b1
    kpp = kp - pre1
    _zero_ref(hb, 16 * NB3)
    lane_off = lane * NB3
    ones = jnp.ones((16,), jnp.int32)

    @plsc.parallel_loop(0, C // 16, unroll=8)
    def _(i):
        e = ev[pl.ds(i * 16, 16)]
        u = lax.bitcast_convert_type(e, jnp.int32)
        m = (u >> 10) == p01
        off = lane_off + (u & (NB3 - 1))
        plsc.addupdate_scatter(hb, [off], ones, mask=m)

    _reduce_banked(hb, hrow, NB3)
    pltpu.sync_copy(hrow, h3_hbm.at[pl.ds(wid * NB3, NB3)])

    @pl.when(wid == 0)
    def _():
        stv[...] = jnp.where(lane == 0, p01, jnp.where(lane == 1, kpp, zeros))
        pltpu.sync_copy(stv, st2_hbm)


@functools.partial(
    pl.kernel,
    mesh=_mesh,
    compiler_params=_params,
    out_type=jax.ShapeDtypeStruct((16,), jnp.float32),
    scratch_types=[
        pltpu.VMEM((NW * NB3,), jnp.int32),
        pltpu.VMEM((16,), jnp.int32),
        pltpu.VMEM((16,), jnp.float32),
    ],
)
def _k4(h3_hbm, st2_hbm, res_hbm, hin, stv, outv):
    wid = _wid()

    @pl.when(wid == 0)
    def _():
        pltpu.sync_copy(h3_hbm, hin)
        pltpu.sync_copy(st2_hbm, stv)
        lane = _lane()
        zeros = jnp.zeros((16,), jnp.int32)
        sv = stv[...]
        p01 = jnp.sum(jnp.where(lane == 0, sv, zeros))
        kpp = jnp.sum(jnp.where(lane == 1, sv, zeros))
        b2, _unused = _reduce_rows_and_scan(hin, NW, NB3, kpp)
        bits = p01 * NB3 + b2
        vec = jnp.zeros((16,), jnp.int32) + bits
        outv[...] = lax.bitcast_convert_type(vec, jnp.float32)
        pltpu.sync_copy(outv, res_hbm)


def kernel(pred, target):
    padp = jnp.concatenate(
        [pred, jnp.full((NPAD - N,), jnp.inf, jnp.float32)])
    padt = jnp.concatenate([target, jnp.zeros((NPAD - N,), jnp.float32)])
    err, h1 = _k1(padp, padt)
    h2, st1 = _k2(err, h1)
    h3, st2 = _k3(err, h2, st1)
    res = _k4(h3, st2)
    return res[0]
